# baseline (speedup 1.0000x reference)
import jax
import jax.numpy as jnp
from jax.experimental import pallas as pl
from jax.experimental.pallas import tpu as pltpu

T = 2048
D = 1024
F = 512
E = 8

_DN = (((1,), (1,)), ((), ()))


def _dense_body(x_ref, wr_ref, wg_ref, wu_ref, wd_ref, o_ref, comb_ref):
    e = pl.program_id(0)
    f = pl.program_id(1)
    xt = x_ref[...]

    @pl.when((e == 0) & (f == 0))
    def _():
        L = jax.lax.dot_general(xt, wr_ref[...], _DN,
                                preferred_element_type=jnp.float32)
        idx = jax.lax.broadcasted_iota(jnp.int32, (T, E), 1)
        m1 = jnp.max(L, axis=1, keepdims=True)
        a1 = jnp.min(jnp.where(L >= m1, idx, E), axis=1, keepdims=True)
        L2 = jnp.where(idx == a1, -1e30, L)
        m2 = jnp.max(L2, axis=1, keepdims=True)
        a2 = jnp.min(jnp.where(L2 >= m2, idx, E), axis=1, keepdims=True)
        e2 = jnp.exp(m2 - m1)
        denom = 1.0 + e2
        g1 = 1.0 / denom
        g2 = e2 / denom
        comb_ref[...] = (jnp.where(idx == a1, g1, 0.0)
                         + jnp.where(idx == a2, g2, 0.0))

    idx = jax.lax.broadcasted_iota(jnp.int32, (T, E), 1)
    c = jnp.sum(comb_ref[...] * (idx == e), axis=1, keepdims=True)
    wg = wg_ref[0]
    wu = wu_ref[0]
    wd = wd_ref[0]
    hg = jax.lax.dot_general(xt, wg, _DN, preferred_element_type=jnp.float32)
    hu = jax.lax.dot_general(xt, wu, _DN, preferred_element_type=jnp.float32)
    h = (hg * jax.lax.logistic(hg)) * hu
    yt = jax.lax.dot_general(h, wd, _DN, preferred_element_type=jnp.float32)
    contrib = c * yt

    @pl.when((e == 0) & (f == 0))
    def _():
        o_ref[...] = contrib

    @pl.when((e > 0) | (f > 0))
    def _():
        o_ref[...] += contrib


def kernel(x, Wr, Wg, Wu, Wd):
    return pl.pallas_call(
        _dense_body,
        grid=(E, 2),
        in_specs=[
            pl.BlockSpec((T, D), lambda j, f: (0, 0)),
            pl.BlockSpec((E, D), lambda j, f: (0, 0)),
            pl.BlockSpec((1, F // 2, D), lambda j, f: (j, f, 0)),
            pl.BlockSpec((1, F // 2, D), lambda j, f: (j, f, 0)),
            pl.BlockSpec((1, D, F // 2), lambda j, f: (j, 0, f)),
        ],
        out_specs=pl.BlockSpec((T, D), lambda j, f: (0, 0)),
        out_shape=jax.ShapeDtypeStruct((T, D), jnp.float32),
        scratch_shapes=[pltpu.VMEM((T, E), jnp.float32)],
        compiler_params=pltpu.CompilerParams(
            dimension_semantics=("arbitrary", "arbitrary")),
    )(x, Wr, Wg, Wu, Wd)


# speedup vs baseline: 1.0776x; 1.0776x over previous
import jax
import jax.numpy as jnp
from jax.experimental import pallas as pl
from jax.experimental.pallas import tpu as pltpu

T = 2048
D = 1024
F = 512
E = 8

_DN = (((1,), (1,)), ((), ()))


def _dense_body(x_ref, wr_ref, wg_ref, wu_ref, wd_ref, o_ref, comb_ref):
    e = pl.program_id(0)
    xt = x_ref[...]

    @pl.when(e == 0)
    def _():
        L = jax.lax.dot_general(xt, wr_ref[...], _DN,
                                preferred_element_type=jnp.float32)
        idx = jax.lax.broadcasted_iota(jnp.int32, (T, E), 1)
        m1 = jnp.max(L, axis=1, keepdims=True)
        a1 = jnp.min(jnp.where(L >= m1, idx, E), axis=1, keepdims=True)
        L2 = jnp.where(idx == a1, -1e30, L)
        m2 = jnp.max(L2, axis=1, keepdims=True)
        a2 = jnp.min(jnp.where(L2 >= m2, idx, E), axis=1, keepdims=True)
        e2 = jnp.exp(m2 - m1)
        denom = 1.0 + e2
        g1 = 1.0 / denom
        g2 = e2 / denom
        comb_ref[...] = (jnp.where(idx == a1, g1, 0.0)
                         + jnp.where(idx == a2, g2, 0.0))

    idx = jax.lax.broadcasted_iota(jnp.int32, (T, E), 1)
    c = jnp.sum(comb_ref[...] * (idx == e), axis=1, keepdims=True)
    wg = wg_ref[0]
    wu = wu_ref[0]
    wd = wd_ref[0]
    hg = jax.lax.dot_general(xt, wg, _DN, preferred_element_type=jnp.float32)
    hu = jax.lax.dot_general(xt, wu, _DN, preferred_element_type=jnp.float32)
    h = (hg * jax.lax.logistic(hg)) * hu
    yt = jax.lax.dot_general(h, wd, _DN, preferred_element_type=jnp.float32)
    contrib = c * yt

    @pl.when(e == 0)
    def _():
        o_ref[...] = contrib

    @pl.when(e > 0)
    def _():
        o_ref[...] += contrib


def kernel(x, Wr, Wg, Wu, Wd):
    return pl.pallas_call(
        _dense_body,
        grid=(E,),
        in_specs=[
            pl.BlockSpec((T, D), lambda j: (0, 0)),
            pl.BlockSpec((E, D), lambda j: (0, 0)),
            pl.BlockSpec((1, F, D), lambda j: (j, 0, 0)),
            pl.BlockSpec((1, F, D), lambda j: (j, 0, 0)),
            pl.BlockSpec((1, D, F), lambda j: (j, 0, 0)),
        ],
        out_specs=pl.BlockSpec((T, D), lambda j: (0, 0)),
        out_shape=jax.ShapeDtypeStruct((T, D), jnp.float32),
        scratch_shapes=[pltpu.VMEM((T, E), jnp.float32)],
        compiler_params=pltpu.CompilerParams(
            dimension_semantics=("arbitrary",)),
    )(x, Wr, Wg, Wu, Wd)
